# baseline (speedup 1.0000x reference)
import jax
import jax.numpy as jnp
from jax.experimental import pallas as pl
from jax.experimental.pallas import tpu as pltpu

N = 50000
M = 16
D_FEAT = 128
D_MSG = 64
BLOCK = 1000


def _body(feat_ref, nts_ref, mts_ref, msg_ref, out_ref):
    w = jnp.exp(-jnp.abs(nts_ref[...] - mts_ref[...]))
    den = jnp.sum(w, axis=1, keepdims=True) + 1e-8
    acc = jnp.zeros((BLOCK, D_MSG), jnp.float32)
    for m in range(M):
        acc = acc + msg_ref[:, m * D_MSG:(m + 1) * D_MSG] * w[:, m:m + 1]
    out_ref[:, :D_FEAT] = feat_ref[...]
    out_ref[:, D_FEAT:] = acc / den


def kernel(target_node_features, node_timestamps, grouped_messages, grouped_message_timestamps):
    msgs2d = grouped_messages.reshape(N, M * D_MSG)
    nts2d = node_timestamps.reshape(N, 1)
    grid = N // BLOCK
    return pl.pallas_call(
        _body,
        grid=(grid,),
        in_specs=[
            pl.BlockSpec((BLOCK, D_FEAT), lambda i: (i, 0)),
            pl.BlockSpec((BLOCK, 1), lambda i: (i, 0)),
            pl.BlockSpec((BLOCK, M), lambda i: (i, 0)),
            pl.BlockSpec((BLOCK, M * D_MSG), lambda i: (i, 0)),
        ],
        out_specs=pl.BlockSpec((BLOCK, D_FEAT + D_MSG), lambda i: (i, 0)),
        out_shape=jax.ShapeDtypeStruct((N, D_FEAT + D_MSG), jnp.float32),
        compiler_params=pltpu.CompilerParams(
            dimension_semantics=("arbitrary",),
        ),
    )(target_node_features, nts2d, grouped_message_timestamps, msgs2d)


# speedup vs baseline: 1.2932x; 1.2932x over previous
import jax
import jax.numpy as jnp
from jax.experimental import pallas as pl
from jax.experimental.pallas import tpu as pltpu

N = 50000
M = 16
D_FEAT = 128
D_MSG = 64
BLOCK = 1000


def _body(feat_ref, nts_ref, mts_ref, msg_ref, out_ref):
    w = jnp.exp(-jnp.abs(nts_ref[...] - mts_ref[...]))
    den = jnp.sum(w, axis=1, keepdims=True) + 1e-8
    col = jax.lax.broadcasted_iota(jnp.int32, (M, M * D_MSG), 1)
    row = jax.lax.broadcasted_iota(jnp.int32, (M, M * D_MSG), 0)
    rep = (col // D_MSG == row).astype(jnp.float32)
    wrep = jax.lax.dot(w, rep, precision=jax.lax.Precision.DEFAULT)
    acc = jnp.zeros((BLOCK, 2 * D_MSG), jnp.float32)
    for k in range(M // 2):
        s = k * 2 * D_MSG
        acc = acc + msg_ref[:, s:s + 2 * D_MSG] * wrep[:, s:s + 2 * D_MSG]
    num = acc[:, :D_MSG] + acc[:, D_MSG:]
    out_ref[:, :D_FEAT] = feat_ref[...]
    out_ref[:, D_FEAT:] = num / den


def kernel(target_node_features, node_timestamps, grouped_messages, grouped_message_timestamps):
    msgs2d = grouped_messages.reshape(N, M * D_MSG)
    nts2d = node_timestamps.reshape(N, 1)
    grid = N // BLOCK
    return pl.pallas_call(
        _body,
        grid=(grid,),
        in_specs=[
            pl.BlockSpec((BLOCK, D_FEAT), lambda i: (i, 0)),
            pl.BlockSpec((BLOCK, 1), lambda i: (i, 0)),
            pl.BlockSpec((BLOCK, M), lambda i: (i, 0)),
            pl.BlockSpec((BLOCK, M * D_MSG), lambda i: (i, 0)),
        ],
        out_specs=pl.BlockSpec((BLOCK, D_FEAT + D_MSG), lambda i: (i, 0)),
        out_shape=jax.ShapeDtypeStruct((N, D_FEAT + D_MSG), jnp.float32),
        compiler_params=pltpu.CompilerParams(
            dimension_semantics=("arbitrary",),
        ),
    )(target_node_features, nts2d, grouped_message_timestamps, msgs2d)


# speedup vs baseline: 1.3066x; 1.0104x over previous
import jax
import jax.numpy as jnp
from jax.experimental import pallas as pl
from jax.experimental.pallas import tpu as pltpu

N = 50000
M = 16
D_FEAT = 128
D_MSG = 64
BLOCK = 2000


def _body(feat_ref, nts_ref, mts_ref, msg_ref, out_ref):
    w = jnp.exp(-jnp.abs(nts_ref[...] - mts_ref[...]))
    den = jnp.sum(w, axis=1, keepdims=True) + 1e-8
    col = jax.lax.broadcasted_iota(jnp.int32, (M, M * D_MSG), 1)
    row = jax.lax.broadcasted_iota(jnp.int32, (M, M * D_MSG), 0)
    rep = (col // D_MSG == row).astype(jnp.float32)
    wrep = jax.lax.dot(w, rep, precision=jax.lax.Precision.DEFAULT)
    acc = jnp.zeros((BLOCK, 2 * D_MSG), jnp.float32)
    for k in range(M // 2):
        s = k * 2 * D_MSG
        acc = acc + msg_ref[:, s:s + 2 * D_MSG] * wrep[:, s:s + 2 * D_MSG]
    num = acc[:, :D_MSG] + acc[:, D_MSG:]
    out_ref[:, :D_FEAT] = feat_ref[...]
    out_ref[:, D_FEAT:] = num / den


def kernel(target_node_features, node_timestamps, grouped_messages, grouped_message_timestamps):
    msgs2d = grouped_messages.reshape(N, M * D_MSG)
    nts2d = node_timestamps.reshape(N, 1)
    grid = N // BLOCK
    return pl.pallas_call(
        _body,
        grid=(grid,),
        in_specs=[
            pl.BlockSpec((BLOCK, D_FEAT), lambda i: (i, 0)),
            pl.BlockSpec((BLOCK, 1), lambda i: (i, 0)),
            pl.BlockSpec((BLOCK, M), lambda i: (i, 0)),
            pl.BlockSpec((BLOCK, M * D_MSG), lambda i: (i, 0)),
        ],
        out_specs=pl.BlockSpec((BLOCK, D_FEAT + D_MSG), lambda i: (i, 0)),
        out_shape=jax.ShapeDtypeStruct((N, D_FEAT + D_MSG), jnp.float32),
        compiler_params=pltpu.CompilerParams(
            dimension_semantics=("arbitrary",),
        ),
    )(target_node_features, nts2d, grouped_message_timestamps, msgs2d)
